# Initial kernel scaffold; baseline (speedup 1.0000x reference)
import jax
import jax.numpy as jnp
from jax.experimental import pallas as pl


def kernel(batch_graph, adj, W1, b1, W2, b2, W3, b3):
    raise NotImplementedError("write your pallas kernel here")


# speedup vs baseline: 22.6616x; 22.6616x over previous
import jax
import jax.numpy as jnp
from jax.experimental import pallas as pl
from functools import partial

B, N, D, H, OUT = 512, 16, 128, 128, 32
G = 16
R = G * N


def _gcn_block(x_ref, a_ref, w1_ref, b1_ref, w2_ref, b2_ref, w3_ref, b3_ref,
               out_ref):
    f32 = jnp.float32
    X = x_ref[...].reshape(R, D)

    ar = a_ref[...].reshape(R, N)
    tiled = jnp.tile(ar, (1, G))
    rg = jax.lax.broadcasted_iota(jnp.int32, (R, R), 0) // N
    cg = jax.lax.broadcasted_iota(jnp.int32, (R, R), 1) // N
    BD = jnp.where(rg == cg, tiled, 0.0)

    mm = partial(jnp.dot, preferred_element_type=f32)

    h = jax.nn.relu(mm(BD, mm(X, w1_ref[...])) + b1_ref[...])
    h = jax.nn.relu(mm(BD, mm(h, w2_ref[...])) + b2_ref[...])
    h = mm(BD, mm(h, w3_ref[...])) + b3_ref[...]

    og = jax.lax.broadcasted_iota(jnp.int32, (G, R), 0)
    cr = jax.lax.broadcasted_iota(jnp.int32, (G, R), 1) // N
    S = jnp.where(og == cr, 1.0 / N, 0.0)
    out_ref[...] = mm(S, h)


def kernel(batch_graph, adj, W1, b1, W2, b2, W3, b3):
    grid = (B // G,)
    out = pl.pallas_call(
        _gcn_block,
        grid=grid,
        in_specs=[
            pl.BlockSpec((G, N, D), lambda i: (i, 0, 0)),
            pl.BlockSpec((G, N, N), lambda i: (i, 0, 0)),
            pl.BlockSpec((D, H), lambda i: (0, 0)),
            pl.BlockSpec((1, H), lambda i: (0, 0)),
            pl.BlockSpec((H, H // 2), lambda i: (0, 0)),
            pl.BlockSpec((1, H // 2), lambda i: (0, 0)),
            pl.BlockSpec((H // 2, OUT), lambda i: (0, 0)),
            pl.BlockSpec((1, OUT), lambda i: (0, 0)),
        ],
        out_specs=pl.BlockSpec((G, OUT), lambda i: (i, 0)),
        out_shape=jax.ShapeDtypeStruct((B, OUT), jnp.float32),
    )(batch_graph, adj, W1, b1.reshape(1, H), W2, b2.reshape(1, H // 2),
      W3, b3.reshape(1, OUT))
    return out.reshape(B, OUT, 1, 1)


# speedup vs baseline: 30.6914x; 1.3543x over previous
import jax
import jax.numpy as jnp
from jax.experimental import pallas as pl
from functools import partial

B, N, D, H, OUT = 512, 16, 128, 128, 32
G = 32
R = G * N


def _gcn_block(x_ref, a_ref, w1_ref, b1_ref, w2_ref, b2_ref, w3_ref, b3_ref,
               out_ref):
    f32 = jnp.float32
    X = x_ref[...].reshape(R, D)

    ar = a_ref[...].reshape(R, N)
    tiled = jnp.tile(ar, (1, G))
    rg = jax.lax.broadcasted_iota(jnp.int32, (R, R), 0) // N
    cg = jax.lax.broadcasted_iota(jnp.int32, (R, R), 1) // N
    BD = jnp.where(rg == cg, tiled, 0.0)

    mm = partial(jnp.dot, preferred_element_type=f32)

    h = jax.nn.relu(mm(BD, mm(X, w1_ref[...])) + b1_ref[...])
    h = jax.nn.relu(mm(BD, mm(h, w2_ref[...])) + b2_ref[...])

    cs = jnp.sum(a_ref[...], axis=1) * (1.0 / N)
    cs_t = jnp.tile(cs, (1, G))
    og = jax.lax.broadcasted_iota(jnp.int32, (G, R), 0)
    cr = jax.lax.broadcasted_iota(jnp.int32, (G, R), 1) // N
    Sc = jnp.where(og == cr, cs_t, 0.0)
    out_ref[...] = mm(mm(Sc, h), w3_ref[...]) + b3_ref[...]


def kernel(batch_graph, adj, W1, b1, W2, b2, W3, b3):
    grid = (B // G,)
    out = pl.pallas_call(
        _gcn_block,
        grid=grid,
        in_specs=[
            pl.BlockSpec((G, N, D), lambda i: (i, 0, 0)),
            pl.BlockSpec((G, N, N), lambda i: (i, 0, 0)),
            pl.BlockSpec((D, H), lambda i: (0, 0)),
            pl.BlockSpec((1, H), lambda i: (0, 0)),
            pl.BlockSpec((H, H // 2), lambda i: (0, 0)),
            pl.BlockSpec((1, H // 2), lambda i: (0, 0)),
            pl.BlockSpec((H // 2, OUT), lambda i: (0, 0)),
            pl.BlockSpec((1, OUT), lambda i: (0, 0)),
        ],
        out_specs=pl.BlockSpec((G, OUT), lambda i: (i, 0)),
        out_shape=jax.ShapeDtypeStruct((B, OUT), jnp.float32),
    )(batch_graph, adj, W1, b1.reshape(1, H), W2, b2.reshape(1, H // 2),
      W3, b3.reshape(1, OUT))
    return out.reshape(B, OUT, 1, 1)


# speedup vs baseline: 33.0481x; 1.0768x over previous
import jax
import jax.numpy as jnp
from jax.experimental import pallas as pl
from functools import partial
import numpy as np

B, N, D, H, OUT = 512, 16, 128, 128, 32
G = 32
R = G * N


def _gcn_block(x_ref, a_ref, bdm_ref, scm_ref, e_ref, w1_ref, b1_ref,
               w2_ref, b2_ref, w3_ref, b3_ref, out_ref):
    f32 = jnp.float32
    mm = partial(jnp.dot, preferred_element_type=f32)

    X = x_ref[...].reshape(R, D)
    ar = a_ref[...].reshape(R, N)

    BD = mm(ar, e_ref[...]) * bdm_ref[...]

    h = jax.nn.relu(mm(BD, mm(X, w1_ref[...])) + b1_ref[...])
    h = jax.nn.relu(mm(BD, mm(h, w2_ref[...])) + b2_ref[...])

    cs = mm(scm_ref[...], ar) * (1.0 / N)
    Sc = mm(cs, e_ref[...]) * scm_ref[...]
    out_ref[...] = mm(mm(Sc, h), w3_ref[...]) + b3_ref[...]


def _masks():
    rg = np.arange(R)[:, None] // N
    cg = np.arange(R)[None, :] // N
    bdm = (rg == cg).astype(np.float32)
    og = np.arange(G)[:, None]
    scm = (og == cg.reshape(1, R)).astype(np.float32)
    e = (np.arange(N)[:, None] == np.arange(R)[None, :] % N)
    return jnp.asarray(bdm), jnp.asarray(scm), jnp.asarray(e, dtype=np.float32)


def kernel(batch_graph, adj, W1, b1, W2, b2, W3, b3):
    bdm, scm, e = _masks()
    grid = (B // G,)
    out = pl.pallas_call(
        _gcn_block,
        grid=grid,
        in_specs=[
            pl.BlockSpec((G, N, D), lambda i: (i, 0, 0)),
            pl.BlockSpec((G, N, N), lambda i: (i, 0, 0)),
            pl.BlockSpec((R, R), lambda i: (0, 0)),
            pl.BlockSpec((G, R), lambda i: (0, 0)),
            pl.BlockSpec((N, R), lambda i: (0, 0)),
            pl.BlockSpec((D, H), lambda i: (0, 0)),
            pl.BlockSpec((1, H), lambda i: (0, 0)),
            pl.BlockSpec((H, H // 2), lambda i: (0, 0)),
            pl.BlockSpec((1, H // 2), lambda i: (0, 0)),
            pl.BlockSpec((H // 2, OUT), lambda i: (0, 0)),
            pl.BlockSpec((1, OUT), lambda i: (0, 0)),
        ],
        out_specs=pl.BlockSpec((G, OUT), lambda i: (i, 0)),
        out_shape=jax.ShapeDtypeStruct((B, OUT), jnp.float32),
    )(batch_graph, adj, bdm, scm, e, W1, b1.reshape(1, H), W2,
      b2.reshape(1, H // 2), W3, b3.reshape(1, OUT))
    return out.reshape(B, OUT, 1, 1)


# speedup vs baseline: 33.1269x; 1.0024x over previous
import jax
import jax.numpy as jnp
from jax.experimental import pallas as pl
from jax.experimental.pallas import tpu as pltpu
from functools import partial
import numpy as np

B, N, D, H, OUT = 512, 16, 128, 128, 32
G = 32
R = G * N


def _gcn_block(x_ref, a_ref, bdm_ref, scm_ref, e_ref, w1_ref, b1_ref,
               w2_ref, b2_ref, w3_ref, b3_ref, out_ref):
    f32 = jnp.float32
    mm = partial(jnp.dot, preferred_element_type=f32)

    X = x_ref[...].reshape(R, D)
    ar = a_ref[...].reshape(R, N)

    BD = mm(ar, e_ref[...]) * bdm_ref[...]

    h = jax.nn.relu(mm(BD, mm(X, w1_ref[...])) + b1_ref[...])
    h = jax.nn.relu(mm(BD, mm(h, w2_ref[...])) + b2_ref[...])

    cs = mm(scm_ref[...], ar) * (1.0 / N)
    Sc = mm(cs, e_ref[...]) * scm_ref[...]
    out_ref[...] = mm(mm(Sc, h), w3_ref[...]) + b3_ref[...]


def _masks():
    rg = np.arange(R)[:, None] // N
    cg = np.arange(R)[None, :] // N
    bdm = (rg == cg).astype(np.float32)
    og = np.arange(G)[:, None]
    scm = (og == cg.reshape(1, R)).astype(np.float32)
    e = (np.arange(N)[:, None] == np.arange(R)[None, :] % N)
    return jnp.asarray(bdm), jnp.asarray(scm), jnp.asarray(e, dtype=np.float32)


def kernel(batch_graph, adj, W1, b1, W2, b2, W3, b3):
    bdm, scm, e = _masks()
    grid = (B // G,)
    out = pl.pallas_call(
        _gcn_block,
        grid=grid,
        in_specs=[
            pl.BlockSpec((G, N, D), lambda i: (i, 0, 0)),
            pl.BlockSpec((G, N, N), lambda i: (i, 0, 0)),
            pl.BlockSpec((R, R), lambda i: (0, 0)),
            pl.BlockSpec((G, R), lambda i: (0, 0)),
            pl.BlockSpec((N, R), lambda i: (0, 0)),
            pl.BlockSpec((D, H), lambda i: (0, 0)),
            pl.BlockSpec((1, H), lambda i: (0, 0)),
            pl.BlockSpec((H, H // 2), lambda i: (0, 0)),
            pl.BlockSpec((1, H // 2), lambda i: (0, 0)),
            pl.BlockSpec((H // 2, OUT), lambda i: (0, 0)),
            pl.BlockSpec((1, OUT), lambda i: (0, 0)),
        ],
        out_specs=pl.BlockSpec((G, OUT), lambda i: (i, 0)),
        out_shape=jax.ShapeDtypeStruct((B, OUT), jnp.float32),
        compiler_params=pltpu.CompilerParams(
            dimension_semantics=("parallel",)),
    )(batch_graph, adj, bdm, scm, e, W1, b1.reshape(1, H), W2,
      b2.reshape(1, H // 2), W3, b3.reshape(1, OUT))
    return out.reshape(B, OUT, 1, 1)
